# initial kernel scaffold (unmeasured)
import jax
import jax.numpy as jnp
from jax import lax
from jax.experimental import pallas as pl
from jax.experimental.pallas import tpu as pltpu

W = 8
NT = 16384
S = NT // W
D = 1024
H = 1024
NE = 64
EL = NE // W
CAP = 204
CST = 256
ROWS = 512
BIG = jnp.int32(1 << 30)


def _a2a(src, offs, rows, *, cid):
    _, c = src.shape

    def body(offs_ref, src_ref, out_ref, send_sems, recv_sems):
        me = lax.axis_index("i")

        bar = pltpu.get_barrier_semaphore()
        for d in range(W):
            @pl.when(me != d)
            def _():
                pl.semaphore_signal(
                    bar, inc=1,
                    device_id=(d,), device_id_type=pl.DeviceIdType.MESH,
                )
        pl.semaphore_wait(bar, W - 1)

        for d in range(W):
            off = offs_ref[d]

            @pl.when(me == d)
            def _():
                out_ref[d] = src_ref[pl.ds(off, rows)]

            @pl.when(me != d)
            def _():
                pltpu.make_async_remote_copy(
                    src_ref=src_ref.at[pl.ds(off, rows)],
                    dst_ref=out_ref.at[me],
                    send_sem=send_sems.at[d],
                    recv_sem=recv_sems.at[me],
                    device_id=(d,),
                    device_id_type=pl.DeviceIdType.MESH,
                ).start()

        for s in range(W):
            @pl.when(me != s)
            def _():
                pltpu.make_async_remote_copy(
                    src_ref=src_ref.at[pl.ds(0, rows)],
                    dst_ref=out_ref.at[s],
                    send_sem=send_sems.at[s],
                    recv_sem=recv_sems.at[s],
                    device_id=(s,),
                    device_id_type=pl.DeviceIdType.MESH,
                ).wait_recv()

        for d in range(W):
            @pl.when(me != d)
            def _():
                pltpu.make_async_remote_copy(
                    src_ref=src_ref.at[pl.ds(0, rows)],
                    dst_ref=out_ref.at[d],
                    send_sem=send_sems.at[d],
                    recv_sem=recv_sems.at[d],
                    device_id=(d,),
                    device_id_type=pl.DeviceIdType.MESH,
                ).wait_send()

    return pl.pallas_call(
        body,
        out_shape=jax.ShapeDtypeStruct((W, rows, c), src.dtype),
        in_specs=[
            pl.BlockSpec(memory_space=pltpu.SMEM),
            pl.BlockSpec(memory_space=pltpu.VMEM),
        ],
        out_specs=pl.BlockSpec(memory_space=pltpu.VMEM),
        scratch_shapes=[
            pltpu.SemaphoreType.DMA((W,)),
            pltpu.SemaphoreType.DMA((W,)),
        ],
        compiler_params=pltpu.CompilerParams(collective_id=cid),
    )(offs, src)


def _moe_matmul(xin, expert_W):

    def body(x_ref, w_ref, o_ref):
        o_ref[...] = jnp.dot(
            x_ref[...], w_ref[0], preferred_element_type=jnp.float32
        )

    return pl.pallas_call(
        body,
        grid=(EL,),
        in_specs=[
            pl.BlockSpec((CST, D), lambda e: (e, 0)),
            pl.BlockSpec((1, D, H), lambda e: (e, 0, 0)),
        ],
        out_specs=pl.BlockSpec((CST, H), lambda e: (e, 0)),
        out_shape=jax.ShapeDtypeStruct((EL * CST, H), jnp.float32),
    )(xin, expert_W)


def kernel(x, router_W, route_idx, expert_W):
    del router_W
    me = lax.axis_index("i")

    rloc = route_idx.reshape(16, 128)
    route = _a2a(rloc, jnp.zeros((W,), jnp.int32), 16, cid=0).reshape(NT)

    perm = jnp.argsort(route, stable=True)
    sorted_e = route[perm]
    starts = jnp.searchsorted(sorted_e, jnp.arange(NE, dtype=sorted_e.dtype))
    rank_sorted = jnp.arange(NT, dtype=jnp.int32) - starts[sorted_e].astype(
        jnp.int32
    )
    rank = jnp.zeros(NT, jnp.int32).at[perm].set(rank_sorted)
    keep = rank < CAP
    gslot = jnp.where(keep, route * CST + rank, BIG)
    tok_of_gslot = (
        jnp.full(NE * CST, -1, jnp.int32)
        .at[gslot]
        .set(jnp.arange(NT, dtype=jnp.int32), mode="drop")
    )

    myroute = route_idx[:, 0]
    mykeep = lax.dynamic_slice(keep, (me * S,), (S,))
    dest = jnp.where(mykeep, myroute // EL, W)
    order = jnp.argsort(dest, stable=True)
    x_pack = jnp.concatenate([x[order], jnp.zeros((ROWS, D), x.dtype)])
    cnts = jnp.bincount(dest, length=W + 1)[:W]
    offs = (jnp.cumsum(cnts) - cnts).astype(jnp.int32)

    xrecv = _a2a(x_pack, offs, ROWS, cid=1)

    keep2 = keep.reshape(W, S)
    route2 = route.reshape(W, S)
    rank2 = rank.reshape(W, S)
    tome = keep2 & ((route2 // EL) == me)
    lslot = (route2 - me * EL) * CST + rank2
    ordr2 = jnp.argsort(~tome, axis=1, stable=True)
    lslot_sorted = jnp.take_along_axis(lslot, ordr2, axis=1)[:, :ROWS]
    nsend = tome.sum(axis=1)
    slot_row = jnp.where(
        jnp.arange(ROWS)[None, :] < nsend[:, None], lslot_sorted, BIG
    )
    xin = (
        jnp.zeros((EL * CST, D), x.dtype)
        .at[slot_row.reshape(-1)]
        .set(xrecv.reshape(W * ROWS, D), mode="drop")
    )

    y = _moe_matmul(xin, expert_W)

    mytoks = lax.dynamic_slice(tok_of_gslot, (me * EL * CST,), (EL * CST,))
    cdest = jnp.where(mytoks >= 0, mytoks // S, W)
    corder = jnp.argsort(cdest, stable=True)
    y_pack = jnp.concatenate([y[corder], jnp.zeros((ROWS, H), y.dtype)])
    ccnts = jnp.bincount(cdest, length=W + 1)[:W]
    coffs = (jnp.cumsum(ccnts) - ccnts).astype(jnp.int32)

    yrecv = _a2a(y_pack, coffs, ROWS, cid=2)

    toks_by_s = tok_of_gslot.reshape(W, EL * CST)
    mine = (toks_by_s >= me * S) & (toks_by_s < (me + 1) * S)
    ordr3 = jnp.argsort(~mine, axis=1, stable=True)
    tok_sorted = jnp.take_along_axis(toks_by_s, ordr3, axis=1)[:, :ROWS] - me * S
    nrecv = mine.sum(axis=1)
    row_tok = jnp.where(
        jnp.arange(ROWS)[None, :] < nrecv[:, None], tok_sorted, BIG
    )
    out = (
        jnp.zeros((S, H), jnp.float32)
        .at[row_tok.reshape(-1)]
        .set(yrecv.reshape(W * ROWS, H), mode="drop")
    )
    return out


# baseline (device time: 750467 ns/iter reference)
import jax
import jax.numpy as jnp
from jax import lax
from jax.experimental import pallas as pl
from jax.experimental.pallas import tpu as pltpu

W = 8
NT = 16384
S = NT // W
D = 1024
H = 1024
NE = 64
EL = NE // W
CAP = 204
CST = 256
ROWS = 512
BIG = jnp.int32(1 << 30)
PACKN = 2048 + 8 * W + ROWS


def _pack_by_group(vals, grp, n_cols):
    cnts = jnp.bincount(grp, length=W + 1)[:W]
    acnts = ((cnts + 7) // 8) * 8
    aoffs = (jnp.cumsum(acnts) - acnts).astype(jnp.int32)
    aoffs_ext = jnp.concatenate([aoffs, jnp.array([BIG], jnp.int32)])
    order = jnp.argsort(grp, stable=True)
    grp_sorted = grp[order]
    gstart = jnp.searchsorted(grp_sorted, jnp.arange(W + 1, dtype=grp.dtype))
    within = jnp.arange(grp.shape[0], dtype=jnp.int32) - gstart[
        grp_sorted
    ].astype(jnp.int32)
    pos = aoffs_ext[grp_sorted] + within
    pack = (
        jnp.zeros((PACKN, n_cols), vals.dtype)
        .at[pos]
        .set(vals[order], mode="drop")
    )
    return pack, aoffs


def _a2a(src, offs, rows, *, cid):
    _, c = src.shape

    def body(offs_ref, src_ref, out_ref, send_sems, recv_sems):
        me = lax.axis_index("i")

        bar = pltpu.get_barrier_semaphore()
        for d in range(W):
            @pl.when(me != d)
            def _():
                pl.semaphore_signal(
                    bar, inc=1,
                    device_id=(d,), device_id_type=pl.DeviceIdType.MESH,
                )
        pl.semaphore_wait(bar, W - 1)

        for d in range(W):
            off = pl.multiple_of(offs_ref[d], 8)

            @pl.when(me == d)
            def _():
                out_ref[d] = src_ref[pl.ds(off, rows)]

            @pl.when(me != d)
            def _():
                pltpu.make_async_remote_copy(
                    src_ref=src_ref.at[pl.ds(off, rows)],
                    dst_ref=out_ref.at[me],
                    send_sem=send_sems.at[d],
                    recv_sem=recv_sems.at[me],
                    device_id=(d,),
                    device_id_type=pl.DeviceIdType.MESH,
                ).start()

        for s in range(W):
            @pl.when(me != s)
            def _():
                pltpu.make_async_remote_copy(
                    src_ref=src_ref.at[pl.ds(0, rows)],
                    dst_ref=out_ref.at[s],
                    send_sem=send_sems.at[s],
                    recv_sem=recv_sems.at[s],
                    device_id=(s,),
                    device_id_type=pl.DeviceIdType.MESH,
                ).wait_recv()

        for d in range(W):
            @pl.when(me != d)
            def _():
                pltpu.make_async_remote_copy(
                    src_ref=src_ref.at[pl.ds(0, rows)],
                    dst_ref=out_ref.at[d],
                    send_sem=send_sems.at[d],
                    recv_sem=recv_sems.at[d],
                    device_id=(d,),
                    device_id_type=pl.DeviceIdType.MESH,
                ).wait_send()

    return pl.pallas_call(
        body,
        out_shape=jax.ShapeDtypeStruct((W, rows, c), src.dtype),
        in_specs=[
            pl.BlockSpec(memory_space=pltpu.SMEM),
            pl.BlockSpec(memory_space=pltpu.VMEM),
        ],
        out_specs=pl.BlockSpec(memory_space=pltpu.VMEM),
        scratch_shapes=[
            pltpu.SemaphoreType.DMA((W,)),
            pltpu.SemaphoreType.DMA((W,)),
        ],
        compiler_params=pltpu.CompilerParams(collective_id=cid),
    )(offs, src)


def _moe_matmul(xin, expert_W):

    def body(x_ref, w_ref, o_ref):
        o_ref[...] = jnp.dot(
            x_ref[...], w_ref[0], preferred_element_type=jnp.float32
        )

    return pl.pallas_call(
        body,
        grid=(EL,),
        in_specs=[
            pl.BlockSpec((CST, D), lambda e: (e, 0)),
            pl.BlockSpec((1, D, H), lambda e: (e, 0, 0)),
        ],
        out_specs=pl.BlockSpec((CST, H), lambda e: (e, 0)),
        out_shape=jax.ShapeDtypeStruct((EL * CST, H), jnp.float32),
    )(xin, expert_W)


def kernel(x, router_W, route_idx, expert_W):
    del router_W
    me = lax.axis_index("i")

    rloc = route_idx.reshape(16, 128)
    route = _a2a(rloc, jnp.zeros((W,), jnp.int32), 16, cid=0).reshape(NT)

    perm = jnp.argsort(route, stable=True)
    sorted_e = route[perm]
    starts = jnp.searchsorted(sorted_e, jnp.arange(NE, dtype=sorted_e.dtype))
    rank_sorted = jnp.arange(NT, dtype=jnp.int32) - starts[sorted_e].astype(
        jnp.int32
    )
    rank = jnp.zeros(NT, jnp.int32).at[perm].set(rank_sorted)
    keep = rank < CAP
    gslot = jnp.where(keep, route * CST + rank, BIG)
    tok_of_gslot = (
        jnp.full(NE * CST, -1, jnp.int32)
        .at[gslot]
        .set(jnp.arange(NT, dtype=jnp.int32), mode="drop")
    )

    myroute = route_idx[:, 0]
    mykeep = lax.dynamic_slice(keep, (me * S,), (S,))
    dest = jnp.where(mykeep, myroute // EL, W)
    x_pack, offs = _pack_by_group(x, dest, D)

    xrecv = _a2a(x_pack, offs, ROWS, cid=1)

    keep2 = keep.reshape(W, S)
    route2 = route.reshape(W, S)
    rank2 = rank.reshape(W, S)
    tome = keep2 & ((route2 // EL) == me)
    lslot = (route2 - me * EL) * CST + rank2
    ordr2 = jnp.argsort(~tome, axis=1, stable=True)
    lslot_sorted = jnp.take_along_axis(lslot, ordr2, axis=1)[:, :ROWS]
    nsend = tome.sum(axis=1)
    slot_row = jnp.where(
        jnp.arange(ROWS)[None, :] < nsend[:, None], lslot_sorted, BIG
    )
    xin = (
        jnp.zeros((EL * CST, D), x.dtype)
        .at[slot_row.reshape(-1)]
        .set(xrecv.reshape(W * ROWS, D), mode="drop")
    )

    y = _moe_matmul(xin, expert_W)

    mytoks = lax.dynamic_slice(tok_of_gslot, (me * EL * CST,), (EL * CST,))
    cdest = jnp.where(mytoks >= 0, mytoks // S, W)
    y_pack, coffs = _pack_by_group(y, cdest, H)

    yrecv = _a2a(y_pack, coffs, ROWS, cid=2)

    toks_by_s = tok_of_gslot.reshape(W, EL * CST)
    mine = (toks_by_s >= me * S) & (toks_by_s < (me + 1) * S)
    ordr3 = jnp.argsort(~mine, axis=1, stable=True)
    tok_sorted = jnp.take_along_axis(toks_by_s, ordr3, axis=1)[:, :ROWS] - me * S
    nrecv = mine.sum(axis=1)
    row_tok = jnp.where(
        jnp.arange(ROWS)[None, :] < nrecv[:, None], tok_sorted, BIG
    )
    out = (
        jnp.zeros((S, H), jnp.float32)
        .at[row_tok.reshape(-1)]
        .set(yrecv.reshape(W * ROWS, H), mode="drop")
    )
    return out
